# Initial kernel scaffold; baseline (speedup 1.0000x reference)
import jax
import jax.numpy as jnp
from jax.experimental import pallas as pl


def kernel(y, x, hidden_states, forward_probs, edge_est, edge_gt, W1, b1, W2, b2, Wn, bn):
    raise NotImplementedError("write your pallas kernel here")


# speedup vs baseline: 2.8219x; 2.8219x over previous
import functools

import jax
import jax.numpy as jnp
from jax.experimental import pallas as pl
from jax.experimental.pallas import tpu as pltpu

N_OBJ = 8
K = 4
X_SIZE = 8
IN_SIZE = X_SIZE + K
MSG_DIM = IN_SIZE * N_OBJ
T_TILE = 128


def _fused_body(feat_ref, w1r_ref, w1s_ref, w2_ref, wnx_ref, wnm_ref,
                b1_ref, b2_ref, bn_ref, out_ref):
    O = N_OBJ
    Tt = T_TILE
    feat = feat_ref[0]
    feat2 = feat.reshape(O * Tt, IN_SIZE)

    a = jnp.dot(feat2, w1r_ref[...], preferred_element_type=jnp.float32)
    bm = jnp.dot(feat2, w1s_ref[...], preferred_element_type=jnp.float32)

    a4 = a.reshape(O, 1, Tt, MSG_DIM)
    b4 = bm.reshape(1, O, Tt, MSG_DIM)
    h1 = jnp.maximum(a4 + b4 + b1_ref[...].reshape(1, 1, 1, MSG_DIM), 0.0)

    h1f = h1.reshape(O * O * Tt, MSG_DIM)
    h2 = jnp.maximum(
        jnp.dot(h1f, w2_ref[...], preferred_element_type=jnp.float32)
        + b2_ref[...], 0.0)
    h2 = h2.reshape(O, O, Tt, MSG_DIM)

    o_idx = jax.lax.broadcasted_iota(jnp.int32, (O, O, Tt, MSG_DIM), 0)
    s_idx = jax.lax.broadcasted_iota(jnp.int32, (O, O, Tt, MSG_DIM), 1)
    h2 = jnp.where(o_idx != s_idx, h2, 0.0)
    agg = jnp.sum(h2, axis=1) * (1.0 / (N_OBJ - 1))
    agg2 = agg.reshape(O * Tt, MSG_DIM)

    out = (jnp.dot(feat2, wnx_ref[...], preferred_element_type=jnp.float32)
           + jnp.dot(agg2, wnm_ref[...], preferred_element_type=jnp.float32)
           + bn_ref[...])
    out_ref[0] = out.reshape(O, Tt, K * K)


@jax.jit
def _run(feat, w1r, w1s, w2t, wnx, wnm, b1, b2, bn):
    B, O, T, _ = feat.shape
    grid = (B, T // T_TILE)
    full = lambda r, c: pl.BlockSpec((r, c), lambda b, t: (0, 0))
    out = pl.pallas_call(
        _fused_body,
        grid=grid,
        in_specs=[
            pl.BlockSpec((1, N_OBJ, T_TILE, IN_SIZE), lambda b, t: (b, 0, t, 0)),
            full(IN_SIZE, MSG_DIM),
            full(IN_SIZE, MSG_DIM),
            full(MSG_DIM, MSG_DIM),
            full(IN_SIZE, K * K),
            full(MSG_DIM, K * K),
            full(1, MSG_DIM),
            full(1, MSG_DIM),
            full(1, K * K),
        ],
        out_specs=pl.BlockSpec((1, N_OBJ, T_TILE, K * K), lambda b, t: (b, 0, t, 0)),
        out_shape=jax.ShapeDtypeStruct((B, O, T, K * K), jnp.float32),
        compiler_params=pltpu.CompilerParams(
            dimension_semantics=("parallel", "parallel")),
    )(feat, w1r, w1s, w2t, wnx, wnm, b1, b2, bn)
    return out.reshape(B, O, T, K, K)


def kernel(y, x, hidden_states, forward_probs, edge_est, edge_gt,
           W1, b1, W2, b2, Wn, bn):
    feat = jnp.concatenate([x, forward_probs], axis=-1)
    w1r = W1[:, :IN_SIZE].T
    w1s = W1[:, IN_SIZE:].T
    w2t = W2.T
    wnx = Wn[:, :IN_SIZE].T
    wnm = Wn[:, IN_SIZE:].T
    return _run(feat, w1r, w1s, w2t, wnx, wnm,
                b1[None, :], b2[None, :], bn[None, :])
